# Initial kernel scaffold; baseline (speedup 1.0000x reference)
import functools

import jax
import jax.numpy as jnp
from jax import lax
from jax.experimental import pallas as pl
from jax.experimental.pallas import tpu as pltpu
from jax.experimental.pallas import tpu_sc as plsc

N = 10000
NPAD = 10016
ROWS_PER_SUB = NPAD // 16
E = 160000
EPAD = 163840
EROWS = EPAD // 128
W = 16
NW = 32
EROWS_PER_W = EROWS // NW
BLK = 8
NBLK = EROWS_PER_W // BLK


def _sc_agg(u, src2d, dst2d, zeros):
    mesh = plsc.VectorSubcoreMesh(core_axis_name="c", subcore_axis_name="s")

    @functools.partial(
        pl.kernel,
        out_type=jax.ShapeDtypeStruct((2, NPAD, W), jnp.float32),
        mesh=mesh,
        scratch_types=[
            pltpu.VMEM((BLK, 128), jnp.int32),
            pltpu.VMEM((BLK, 128), jnp.int32),
            pltpu.VMEM((128, W), jnp.float32),
            pltpu.VMEM_SHARED((NPAD, W), jnp.float32),
        ],
    )
    def agg(u_hbm, src_hbm, dst_hbm, z_hbm, parts_hbm, idx_s, idx_d, rows, acc):
        cid = lax.axis_index("c")
        sid = lax.axis_index("s")
        wid = sid * 2 + cid
        stripe = pl.ds(sid * ROWS_PER_SUB, ROWS_PER_SUB)

        pltpu.sync_copy(z_hbm.at[stripe], acc.at[stripe])
        plsc.subcore_barrier()

        @pl.loop(0, NBLK)
        def _(c):
            rowbase = wid * EROWS_PER_W + c * BLK
            pltpu.sync_copy(src_hbm.at[pl.ds(rowbase, BLK)], idx_s)
            pltpu.sync_copy(dst_hbm.at[pl.ds(rowbase, BLK)], idx_d)
            for j in range(BLK):
                pltpu.sync_copy(u_hbm.at[idx_s.at[j]], rows)
                pltpu.sync_copy(rows, acc.at[idx_d.at[j]], add=True)

        plsc.subcore_barrier()
        pltpu.sync_copy(acc.at[stripe], parts_hbm.at[cid, stripe])

    return agg(u, src2d, dst2d, zeros)


def _tc_call(body, out_shapes, *args):
    return pl.pallas_call(body, out_shape=out_shapes)(*args)


def _mm_body(x_ref, w_ref, o_ref):
    o_ref[...] = jnp.dot(x_ref[...], w_ref[...],
                         preferred_element_type=jnp.float32)


def _deg_body(dp_ref, z_ref, dinv_ref, u_ref):
    deg = dp_ref[0] + dp_ref[1] + 1.0
    dinv = lax.rsqrt(deg)
    dinv_ref[...] = dinv
    u_ref[...] = dinv * z_ref[...]


def _layer_body(p_ref, u_ref, dinv_ref, w_ref, b_ref, un_ref):
    h = jnp.tanh(dinv_ref[...] * (p_ref[0] + p_ref[1] + u_ref[...])
                 + b_ref[...])
    un_ref[...] = dinv_ref[...] * jnp.dot(h, w_ref[...],
                                          preferred_element_type=jnp.float32)


def _final_body(p_ref, u_ref, dinv_ref, b_ref, wc_ref, bc_ref, out_ref, h_ref):
    h = jnp.tanh(dinv_ref[...] * (p_ref[0] + p_ref[1] + u_ref[...])
                 + b_ref[...])
    h_ref[...] = h
    out_ref[...] = jnp.dot(h, wc_ref[...],
                           preferred_element_type=jnp.float32) + bc_ref[...]


def _padw(w):
    return jnp.pad(w, ((0, 16 - w.shape[0]), (0, 16 - w.shape[1])))


def kernel(x, edge_index, W1, b1, W2, b2, W3, b3, Wc, bc):
    f32 = jnp.float32
    src = edge_index[0]
    dst = edge_index[1]
    pad = jnp.full((EPAD - E,), N, dtype=jnp.int32)
    src2d = jnp.concatenate([src, pad]).reshape(EROWS, 128)
    dst2d = jnp.concatenate([dst, pad]).reshape(EROWS, 128)

    xp = jnp.pad(x, ((0, NPAD - N), (0, 0)))
    W1p = jnp.pad(W1, ((0, 0), (0, W - W1.shape[1])))
    W2p = _padw(W2)
    W3p = _padw(W3)
    Wcp = jnp.pad(Wc, ((0, 16 - Wc.shape[0]), (0, 0)))
    b1p = jnp.pad(b1, (0, W - b1.shape[0])).reshape(1, W)
    b2p = jnp.pad(b2, (0, W - b2.shape[0])).reshape(1, W)
    b3p = jnp.pad(b3, (0, W - b3.shape[0])).reshape(1, W)
    bcp = bc.reshape(1, bc.shape[0])

    zeros = jnp.zeros((NPAD, W), f32)
    ones = jnp.ones((NPAD, W), f32)

    sds = jax.ShapeDtypeStruct

    z1 = _tc_call(_mm_body, sds((NPAD, W), f32), xp, W1p)

    degp = _sc_agg(ones, dst2d, dst2d, zeros)
    dinv, u1 = _tc_call(_deg_body, [sds((NPAD, W), f32), sds((NPAD, W), f32)],
                        degp, z1)

    p1 = _sc_agg(u1, src2d, dst2d, zeros)
    u2 = _tc_call(_layer_body, sds((NPAD, W), f32), p1, u1, dinv, W2p, b1p)

    p2 = _sc_agg(u2, src2d, dst2d, zeros)
    u3 = _tc_call(_layer_body, sds((NPAD, W), f32), p2, u2, dinv, W3p, b2p)

    p3 = _sc_agg(u3, src2d, dst2d, zeros)
    out16, h16 = _tc_call(
        _final_body,
        [sds((NPAD, bc.shape[0]), f32), sds((NPAD, W), f32)],
        p3, u3, dinv, b3p, Wcp, bcp)

    return (out16[:N], h16[:N, :2])


# speedup vs baseline: 16.4496x; 16.4496x over previous
import functools

import jax
import jax.numpy as jnp
from jax import lax
from jax.experimental import pallas as pl
from jax.experimental.pallas import tpu as pltpu
from jax.experimental.pallas import tpu_sc as plsc

N = 10000
NPAD = 10112
ROWS_PER_SUB = NPAD // 16
E = 160000
EPAD = 163840
EROWS = EPAD // 128
W = 16
NW = 32
EROWS_PER_W = EROWS // NW
BLK = 8
NBLK = EROWS_PER_W // BLK


def _sc_agg(u, src2d, dst2d, zeros):
    mesh = plsc.VectorSubcoreMesh(core_axis_name="c", subcore_axis_name="s")

    @functools.partial(
        pl.kernel,
        out_type=jax.ShapeDtypeStruct((2, NPAD, W), jnp.float32),
        mesh=mesh,
        compiler_params=pltpu.CompilerParams(use_tc_tiling_on_sc=False),
        scratch_types=[
            pltpu.VMEM((BLK, 128), jnp.int32),
            pltpu.VMEM((BLK, 128), jnp.int32),
            pltpu.VMEM((128, W), jnp.float32),
            pltpu.VMEM_SHARED((NPAD, W), jnp.float32),
        ],
    )
    def agg(u_hbm, src_hbm, dst_hbm, z_hbm, parts_hbm, idx_s, idx_d, rows, acc):
        cid = lax.axis_index("c")
        sid = lax.axis_index("s")
        wid = sid * 2 + cid
        stripe = pl.ds(sid * ROWS_PER_SUB, ROWS_PER_SUB)

        pltpu.sync_copy(z_hbm.at[stripe], acc.at[stripe])
        plsc.subcore_barrier()

        @pl.loop(0, NBLK)
        def _(c):
            rowbase = wid * EROWS_PER_W + c * BLK
            pltpu.sync_copy(src_hbm.at[pl.ds(rowbase, BLK)], idx_s)
            pltpu.sync_copy(dst_hbm.at[pl.ds(rowbase, BLK)], idx_d)
            for j in range(BLK):
                pltpu.sync_copy(u_hbm.at[idx_s.at[j]], rows)
                pltpu.sync_copy(rows, acc.at[idx_d.at[j]], add=True)

        plsc.subcore_barrier()
        pltpu.sync_copy(acc.at[stripe], parts_hbm.at[cid, stripe])

    return agg(u, src2d, dst2d, zeros)


def _tc_call(body, out_shapes, *args):
    return pl.pallas_call(body, out_shape=out_shapes)(*args)


def _mm_body(x_ref, w_ref, o_ref):
    o_ref[...] = jnp.dot(x_ref[...], w_ref[...],
                         preferred_element_type=jnp.float32)


def _deg_body(dp_ref, z_ref, dinv_ref, u_ref):
    deg = dp_ref[0] + dp_ref[1] + 1.0
    dinv = lax.rsqrt(deg)
    dinv_ref[...] = dinv
    u_ref[...] = dinv * z_ref[...]


def _layer_body(p_ref, u_ref, dinv_ref, w_ref, b_ref, un_ref):
    h = jnp.tanh(dinv_ref[...] * (p_ref[0] + p_ref[1] + u_ref[...])
                 + b_ref[...])
    un_ref[...] = dinv_ref[...] * jnp.dot(h, w_ref[...],
                                          preferred_element_type=jnp.float32)


def _final_body(p_ref, u_ref, dinv_ref, b_ref, wc_ref, bc_ref, out_ref, h_ref):
    h = jnp.tanh(dinv_ref[...] * (p_ref[0] + p_ref[1] + u_ref[...])
                 + b_ref[...])
    h_ref[...] = h
    out_ref[...] = jnp.dot(h, wc_ref[...],
                           preferred_element_type=jnp.float32) + bc_ref[...]


def _padw(w):
    return jnp.pad(w, ((0, 16 - w.shape[0]), (0, 16 - w.shape[1])))


def kernel(x, edge_index, W1, b1, W2, b2, W3, b3, Wc, bc):
    f32 = jnp.float32
    src = edge_index[0]
    dst = edge_index[1]
    pad = jnp.full((EPAD - E,), N, dtype=jnp.int32)
    src2d = jnp.concatenate([src, pad]).reshape(EROWS, 128)
    dst2d = jnp.concatenate([dst, pad]).reshape(EROWS, 128)

    xp = jnp.pad(x, ((0, NPAD - N), (0, 0)))
    W1p = jnp.pad(W1, ((0, 0), (0, W - W1.shape[1])))
    W2p = _padw(W2)
    W3p = _padw(W3)
    Wcp = jnp.pad(Wc, ((0, 16 - Wc.shape[0]), (0, 0)))
    b1p = jnp.pad(b1, (0, W - b1.shape[0])).reshape(1, W)
    b2p = jnp.pad(b2, (0, W - b2.shape[0])).reshape(1, W)
    b3p = jnp.pad(b3, (0, W - b3.shape[0])).reshape(1, W)
    bcp = bc.reshape(1, bc.shape[0])

    zeros = jnp.zeros((NPAD, W), f32)
    ones = jnp.ones((NPAD, W), f32)

    sds = jax.ShapeDtypeStruct

    z1 = _tc_call(_mm_body, sds((NPAD, W), f32), xp, W1p)

    degp = _sc_agg(ones, dst2d, dst2d, zeros)
    dinv, u1 = _tc_call(_deg_body, [sds((NPAD, W), f32), sds((NPAD, W), f32)],
                        degp, z1)

    p1 = _sc_agg(u1, src2d, dst2d, zeros)
    u2 = _tc_call(_layer_body, sds((NPAD, W), f32), p1, u1, dinv, W2p, b1p)

    p2 = _sc_agg(u2, src2d, dst2d, zeros)
    u3 = _tc_call(_layer_body, sds((NPAD, W), f32), p2, u2, dinv, W3p, b2p)

    p3 = _sc_agg(u3, src2d, dst2d, zeros)
    out16, h16 = _tc_call(
        _final_body,
        [sds((NPAD, bc.shape[0]), f32), sds((NPAD, W), f32)],
        p3, u3, dinv, b3p, Wcp, bcp)

    return (out16[:N], h16[:N, :2])


# speedup vs baseline: 21.4227x; 1.3023x over previous
import functools

import jax
import jax.numpy as jnp
from jax import lax
from jax.experimental import pallas as pl
from jax.experimental.pallas import tpu as pltpu
from jax.experimental.pallas import tpu_sc as plsc

N = 10000
NPAD = 10112
ROWS_PER_SUB = NPAD // 16
E = 160000
EPAD = 163840
EROWS = EPAD // 128
W = 16
NW = 32
EROWS_PER_W = EROWS // NW
BLK = 8
NBLK = EROWS_PER_W // BLK


def _sc_agg(u, src2d, dst2d, zeros):
    mesh = plsc.VectorSubcoreMesh(core_axis_name="c", subcore_axis_name="s")

    @functools.partial(
        pl.kernel,
        out_type=jax.ShapeDtypeStruct((2, NPAD, W), jnp.float32),
        mesh=mesh,
        compiler_params=pltpu.CompilerParams(use_tc_tiling_on_sc=False),
        scratch_types=[
            pltpu.VMEM((EROWS_PER_W, 128), jnp.int32),
            pltpu.VMEM((EROWS_PER_W, 128), jnp.int32),
            pltpu.VMEM((EROWS_PER_W * 128, W), jnp.float32),
            pltpu.VMEM_SHARED((NPAD, W), jnp.float32),
            pltpu.SemaphoreType.DMA,
            pltpu.SemaphoreType.DMA,
            pltpu.SemaphoreType.DMA,
        ],
    )
    def agg(u_hbm, src_hbm, dst_hbm, z_hbm, parts_hbm, idx_s, idx_d, rows,
            acc, sem_i, sem_g, sem_s):
        cid = lax.axis_index("c")
        sid = lax.axis_index("s")
        wid = sid * 2 + cid
        stripe = pl.ds(sid * ROWS_PER_SUB, ROWS_PER_SUB)
        erow0 = wid * EROWS_PER_W

        hz = pltpu.async_copy(z_hbm.at[stripe], acc.at[stripe], sem_i)
        hs = pltpu.async_copy(src_hbm.at[pl.ds(erow0, EROWS_PER_W)], idx_s,
                              sem_i)
        hd = pltpu.async_copy(dst_hbm.at[pl.ds(erow0, EROWS_PER_W)], idx_d,
                              sem_i)
        hz.wait()
        hs.wait()
        hd.wait()
        plsc.subcore_barrier()

        @pl.loop(0, EROWS_PER_W)
        def _(j):
            pltpu.async_copy(u_hbm.at[idx_s.at[j]],
                             rows.at[pl.ds(j * 128, 128)], sem_g)

        @pl.loop(0, EROWS_PER_W)
        def _(j):
            pltpu.make_async_copy(u_hbm.at[idx_s.at[0]],
                                  rows.at[pl.ds(0, 128)], sem_g).wait()

        @pl.loop(0, EROWS_PER_W)
        def _(j):
            pltpu.async_copy(rows.at[pl.ds(j * 128, 128)],
                             acc.at[idx_d.at[j]], sem_s, add=True)

        @pl.loop(0, EROWS_PER_W)
        def _(j):
            pltpu.make_async_copy(rows.at[pl.ds(0, 128)],
                                  acc.at[idx_d.at[0]], sem_s).wait()

        plsc.subcore_barrier()
        pltpu.sync_copy(acc.at[stripe], parts_hbm.at[cid, stripe])

    return agg(u, src2d, dst2d, zeros)


def _tc_call(body, out_shapes, *args):
    return pl.pallas_call(body, out_shape=out_shapes)(*args)


def _mm_body(x_ref, w_ref, o_ref):
    o_ref[...] = jnp.dot(x_ref[...], w_ref[...],
                         preferred_element_type=jnp.float32)


def _deg_body(dp_ref, z_ref, dinv_ref, u_ref):
    deg = dp_ref[0] + dp_ref[1] + 1.0
    dinv = lax.rsqrt(deg)
    dinv_ref[...] = dinv
    u_ref[...] = dinv * z_ref[...]


def _layer_body(p_ref, u_ref, dinv_ref, w_ref, b_ref, un_ref):
    h = jnp.tanh(dinv_ref[...] * (p_ref[0] + p_ref[1] + u_ref[...])
                 + b_ref[...])
    un_ref[...] = dinv_ref[...] * jnp.dot(h, w_ref[...],
                                          preferred_element_type=jnp.float32)


def _final_body(p_ref, u_ref, dinv_ref, b_ref, wc_ref, bc_ref, out_ref, h_ref):
    h = jnp.tanh(dinv_ref[...] * (p_ref[0] + p_ref[1] + u_ref[...])
                 + b_ref[...])
    h_ref[...] = h
    out_ref[...] = jnp.dot(h, wc_ref[...],
                           preferred_element_type=jnp.float32) + bc_ref[...]


def _padw(w):
    return jnp.pad(w, ((0, 16 - w.shape[0]), (0, 16 - w.shape[1])))


def kernel(x, edge_index, W1, b1, W2, b2, W3, b3, Wc, bc):
    f32 = jnp.float32
    src = edge_index[0]
    dst = edge_index[1]
    pad = jnp.full((EPAD - E,), N, dtype=jnp.int32)
    src2d = jnp.concatenate([src, pad]).reshape(EROWS, 128)
    dst2d = jnp.concatenate([dst, pad]).reshape(EROWS, 128)

    xp = jnp.pad(x, ((0, NPAD - N), (0, 0)))
    W1p = jnp.pad(W1, ((0, 0), (0, W - W1.shape[1])))
    W2p = _padw(W2)
    W3p = _padw(W3)
    Wcp = jnp.pad(Wc, ((0, 16 - Wc.shape[0]), (0, 0)))
    b1p = jnp.pad(b1, (0, W - b1.shape[0])).reshape(1, W)
    b2p = jnp.pad(b2, (0, W - b2.shape[0])).reshape(1, W)
    b3p = jnp.pad(b3, (0, W - b3.shape[0])).reshape(1, W)
    bcp = bc.reshape(1, bc.shape[0])

    zeros = jnp.zeros((NPAD, W), f32)
    ones = jnp.ones((NPAD, W), f32)

    sds = jax.ShapeDtypeStruct

    z1 = _tc_call(_mm_body, sds((NPAD, W), f32), xp, W1p)

    degp = _sc_agg(ones, dst2d, dst2d, zeros)
    dinv, u1 = _tc_call(_deg_body, [sds((NPAD, W), f32), sds((NPAD, W), f32)],
                        degp, z1)

    p1 = _sc_agg(u1, src2d, dst2d, zeros)
    u2 = _tc_call(_layer_body, sds((NPAD, W), f32), p1, u1, dinv, W2p, b1p)

    p2 = _sc_agg(u2, src2d, dst2d, zeros)
    u3 = _tc_call(_layer_body, sds((NPAD, W), f32), p2, u2, dinv, W3p, b2p)

    p3 = _sc_agg(u3, src2d, dst2d, zeros)
    out16, h16 = _tc_call(
        _final_body,
        [sds((NPAD, bc.shape[0]), f32), sds((NPAD, W), f32)],
        p3, u3, dinv, b3p, Wcp, bcp)

    return (out16[:N], h16[:N, :2])


# speedup vs baseline: 33.1923x; 1.5494x over previous
import functools

import jax
import jax.numpy as jnp
from jax import lax
from jax.experimental import pallas as pl
from jax.experimental.pallas import tpu as pltpu
from jax.experimental.pallas import tpu_sc as plsc

N = 10000
NPAD = 10112
ROWS_PER_SUB = NPAD // 16
E = 160000
EROWS = E // 128
NW = 32
EROWS_PER_W = 39
TAIL = 8
W = 16
BLK = 13
NBLK = EROWS_PER_W // BLK


def _sc_agg(u, ei3, zeros):
    mesh = plsc.VectorSubcoreMesh(core_axis_name="c", subcore_axis_name="s")

    @functools.partial(
        pl.kernel,
        out_type=jax.ShapeDtypeStruct((2, NPAD, W), jnp.float32),
        mesh=mesh,
        compiler_params=pltpu.CompilerParams(use_tc_tiling_on_sc=False),
        scratch_types=[
            pltpu.VMEM((EROWS_PER_W, 128), jnp.int32),
            pltpu.VMEM((EROWS_PER_W, 128), jnp.int32),
            pltpu.VMEM((TAIL,), jnp.int32),
            pltpu.VMEM((TAIL,), jnp.int32),
            pltpu.VMEM((EROWS_PER_W * 128, W), jnp.float32),
            pltpu.VMEM((TAIL, W), jnp.float32),
            pltpu.VMEM_SHARED((NPAD, W), jnp.float32),
            pltpu.SemaphoreType.DMA,
            pltpu.SemaphoreType.DMA((2,)),
            pltpu.SemaphoreType.DMA((2,)),
        ],
    )
    def agg(u_hbm, ei_hbm, z_hbm, parts_hbm, idx_s, idx_d, tidx_s, tidx_d,
            rows, trows, acc, sem_i, sem_g, sem_s):
        cid = lax.axis_index("c")
        sid = lax.axis_index("s")
        wid = sid * 2 + cid
        stripe = pl.ds(sid * ROWS_PER_SUB, ROWS_PER_SUB)
        erow0 = wid * EROWS_PER_W
        trow = EROWS - 2 + wid // 16
        tcol = lax.rem(wid, 16) * TAIL

        h0 = pltpu.async_copy(z_hbm.at[stripe], acc.at[stripe], sem_i)
        h1 = pltpu.async_copy(ei_hbm.at[0, pl.ds(erow0, EROWS_PER_W)], idx_s,
                              sem_i)
        h2 = pltpu.async_copy(ei_hbm.at[1, pl.ds(erow0, EROWS_PER_W)], idx_d,
                              sem_i)
        h3 = pltpu.async_copy(ei_hbm.at[0, trow, pl.ds(tcol, TAIL)], tidx_s,
                              sem_i)
        h4 = pltpu.async_copy(ei_hbm.at[1, trow, pl.ds(tcol, TAIL)], tidx_d,
                              sem_i)
        h0.wait(); h1.wait(); h2.wait(); h3.wait(); h4.wait()
        plsc.subcore_barrier()

        def fire_gathers(g, slot):
            @pl.loop(0, BLK)
            def _(i):
                j = g * BLK + i
                pltpu.async_copy(u_hbm.at[idx_s.at[j]],
                                 rows.at[pl.ds(j * 128, 128)],
                                 sem_g.at[slot])

        def drain_gathers(slot):
            @pl.loop(0, BLK)
            def _(i):
                pltpu.make_async_copy(u_hbm.at[idx_s.at[0]],
                                      rows.at[pl.ds(0, 128)],
                                      sem_g.at[slot]).wait()

        def fire_scatters(g, slot):
            @pl.loop(0, BLK)
            def _(i):
                j = g * BLK + i
                pltpu.async_copy(rows.at[pl.ds(j * 128, 128)],
                                 acc.at[idx_d.at[j]], sem_s.at[slot],
                                 add=True)

        def drain_scatters(slot):
            @pl.loop(0, BLK)
            def _(i):
                pltpu.make_async_copy(rows.at[pl.ds(0, 128)],
                                      acc.at[idx_d.at[0]],
                                      sem_s.at[slot]).wait()

        fire_gathers(0, 0)

        @pl.loop(0, NBLK)
        def _(g):
            slot = lax.rem(g, 2)
            nslot = lax.rem(g + 1, 2)

            @pl.when(g + 1 < NBLK)
            def _():
                fire_gathers(g + 1, nslot)

            drain_gathers(slot)

            @pl.when(g >= 2)
            def _():
                drain_scatters(slot)

            fire_scatters(g, slot)

        pltpu.sync_copy(u_hbm.at[tidx_s], trows)
        pltpu.sync_copy(trows, acc.at[tidx_d], add=True)

        drain_scatters(lax.rem(NBLK - 2, 2))
        drain_scatters(lax.rem(NBLK - 1, 2))

        plsc.subcore_barrier()
        pltpu.sync_copy(acc.at[stripe], parts_hbm.at[cid, stripe])

    return agg(u, ei3, zeros)


def _tc_call(body, out_shapes, *args):
    return pl.pallas_call(body, out_shape=out_shapes)(*args)


def _mm_body(x_ref, w_ref, o_ref):
    o_ref[0:N, :] = jnp.dot(x_ref[...], w_ref[...],
                            preferred_element_type=jnp.float32)
    o_ref[N:NPAD, :] = jnp.zeros((NPAD - N, W), jnp.float32)


def _deg_body(dp_ref, z_ref, dinv_ref, u_ref):
    deg = dp_ref[0] + dp_ref[1] + 1.0
    dinv = lax.rsqrt(deg)
    dinv_ref[...] = dinv
    u_ref[...] = dinv * z_ref[...]


def _layer_body(p_ref, u_ref, dinv_ref, w_ref, b_ref, un_ref):
    h = jnp.tanh(dinv_ref[...] * (p_ref[0] + p_ref[1] + u_ref[...])
                 + b_ref[...])
    un_ref[...] = dinv_ref[...] * jnp.dot(h, w_ref[...],
                                          preferred_element_type=jnp.float32)


def _final_body(p_ref, u_ref, dinv_ref, b_ref, wc_ref, bc_ref, out_ref, h_ref):
    h = jnp.tanh(dinv_ref[...] * (p_ref[0] + p_ref[1] + u_ref[...])
                 + b_ref[...])
    h_ref[...] = h[0:N, 0:2]
    out_ref[...] = jnp.dot(h[0:N], wc_ref[...],
                           preferred_element_type=jnp.float32) + bc_ref[...]


def _padw(w):
    return jnp.pad(w, ((0, 16 - w.shape[0]), (0, 16 - w.shape[1])))


def kernel(x, edge_index, W1, b1, W2, b2, W3, b3, Wc, bc):
    f32 = jnp.float32
    ei3 = edge_index.reshape(2, EROWS, 128)

    W1p = jnp.pad(W1, ((0, 0), (0, W - W1.shape[1])))
    W2p = _padw(W2)
    W3p = _padw(W3)
    Wcp = jnp.pad(Wc, ((0, 16 - Wc.shape[0]), (0, 0)))
    b1p = jnp.pad(b1, (0, W - b1.shape[0])).reshape(1, W)
    b2p = jnp.pad(b2, (0, W - b2.shape[0])).reshape(1, W)
    b3p = jnp.pad(b3, (0, W - b3.shape[0])).reshape(1, W)
    bcp = bc.reshape(1, bc.shape[0])

    zeros = jnp.zeros((NPAD, W), f32)
    ones = jnp.ones((NPAD, W), f32)

    sds = jax.ShapeDtypeStruct

    z1 = _tc_call(_mm_body, sds((NPAD, W), f32), x, W1p)

    degp = _sc_agg(ones, ei3, zeros)
    dinv, u1 = _tc_call(_deg_body, [sds((NPAD, W), f32), sds((NPAD, W), f32)],
                        degp, z1)

    p1 = _sc_agg(u1, ei3, zeros)
    u2 = _tc_call(_layer_body, sds((NPAD, W), f32), p1, u1, dinv, W2p, b1p)

    p2 = _sc_agg(u2, ei3, zeros)
    u3 = _tc_call(_layer_body, sds((NPAD, W), f32), p2, u2, dinv, W3p, b2p)

    p3 = _sc_agg(u3, ei3, zeros)
    out, h = _tc_call(
        _final_body,
        [sds((N, bc.shape[0]), f32), sds((N, 2), f32)],
        p3, u3, dinv, b3p, Wcp, bcp)

    return (out, h)
